# baseline (speedup 1.0000x reference)
import functools

import jax
import jax.numpy as jnp
from jax import lax
from jax.experimental import pallas as pl
from jax.experimental.pallas import tpu as pltpu
from jax.experimental.pallas import tpu_sc as plsc

_K = 16
_D = 128


def _silu(x):
    return x * (1.0 / (1.0 + jnp.exp(-x)))


def _knn_prep(xk, xr):
    sq = jnp.sum(xk * xk, axis=1, keepdims=True)
    sqr = jnp.sum(xr * xr, axis=1, keepdims=True)
    inner = lax.dot_general(xr, xk, (((1,), (1,)), ((), ())),
                            preferred_element_type=jnp.float32)
    d2 = sqr + jnp.transpose(sq) - 2.0 * inner
    return jnp.sqrt(jnp.maximum(d2, 0.0))


def _sum_sq_seq(xd, nch):
    acc = xd[:, 0:1] * xd[:, 0:1]
    for i in range(1, nch):
        acc = acc + xd[:, i:i + 1] * xd[:, i:i + 1]
    return acc


def _pick_and_mask(d_ref, iota, N):
    dc = d_ref[...]
    m = jnp.max(dc, axis=1, keepdims=True)
    pick = jnp.min(jnp.where(dc == m, iota, N), axis=1, keepdims=True)
    d_ref[...] = jnp.where(iota == pick, -jnp.inf, dc)
    return m, pick


def _edge1_body(xk_ref, gath_ref, e1wT_ref, xwT_ref, fwT_ref, misc_ref,
                xs_out_ref, f_out_ref, *, R):
    xr = xk_ref[0]
    eb = misc_ref[0:1, :]
    xb = misc_ref[2:3, 0:8]
    fb = misc_ref[3:4, :]

    s = jnp.zeros((R, 64), jnp.float32)
    xacc = jnp.zeros((R, 8), jnp.float32)
    zr1 = jnp.zeros((R, 1), jnp.float32)
    on1 = jnp.ones((R, 1), jnp.float32)
    zr5 = jnp.zeros((R, 5), jnp.float32)
    zr2 = jnp.zeros((R, 2), jnp.float32)
    for k in range(_K):
        gk = gath_ref[0, :, k, :]
        xd = gk[:, 64:72] - xr
        xdsq = _sum_sq_seq(xd, 3)
        feat = jnp.concatenate([zr1, on1, xdsq, zr5], axis=1)
        mj = _silu(jnp.dot(feat, e1wT_ref[...],
                           preferred_element_type=jnp.float32) + eb)
        s = s + mj
        phi = _silu(jnp.dot(mj, xwT_ref[...],
                            preferred_element_type=jnp.float32) + xb)
        xd3 = xd[:, 0:3]
        xdc = jnp.concatenate([xd3, xd3, zr2], axis=1)
        p0 = phi[:, 0:1]
        p1 = phi[:, 1:2]
        phic = jnp.concatenate([p0, p0, p0, p1, p1, p1, p0, p0], axis=1)
        xacc = xacc + xdc * phic

    xr3 = xr[:, 0:3]
    xs_base = jnp.concatenate([xr3, xr3, zr2], axis=1)
    xs_out_ref[0] = xs_base + xacc * (1.0 / _K)
    zr7 = jnp.zeros((R, 7), jnp.float32)
    f_in = jnp.concatenate([on1, s, zr7], axis=1)
    f_out_ref[0] = _silu(jnp.dot(f_in, fwT_ref[...],
                                 preferred_element_type=jnp.float32) + fb)


def _call_edge1(xk, gath, e1wT, xwT, fwT, misc, R):
    B, N, _ = xk.shape
    body = functools.partial(_edge1_body, R=R)
    return pl.pallas_call(
        body,
        grid=(B, N // R),
        in_specs=[
            pl.BlockSpec((1, R, 8), lambda b, rb: (b, rb, 0)),
            pl.BlockSpec((1, R, _K, _D), lambda b, rb: (b, rb, 0, 0)),
            pl.BlockSpec((8, 64), lambda b, rb: (0, 0)),
            pl.BlockSpec((64, 8), lambda b, rb: (0, 0)),
            pl.BlockSpec((72, 64), lambda b, rb: (0, 0)),
            pl.BlockSpec((8, 64), lambda b, rb: (0, 0)),
        ],
        out_specs=[
            pl.BlockSpec((1, R, 8), lambda b, rb: (b, rb, 0)),
            pl.BlockSpec((1, R, 64), lambda b, rb: (b, rb, 0)),
        ],
        out_shape=[
            jax.ShapeDtypeStruct((B, N, 8), jnp.float32),
            jax.ShapeDtypeStruct((B, N, 64), jnp.float32),
        ],
        compiler_params=pltpu.CompilerParams(
            dimension_semantics=("parallel", "arbitrary")),
    )(xk, gath, e1wT, xwT, fwT, misc)


def _topk_body(xk_ref, f_ref, idx_out, tab_out, d_ref, *, R, N):
    b = pl.program_id(0)
    rb = pl.program_id(1)
    xk = xk_ref[0]
    r0 = pl.multiple_of(rb * R, R)
    xr = xk_ref[0, pl.ds(r0, R), :]
    fr = f_ref[0, pl.ds(r0, R), :]
    d_ref[...] = _knn_prep(xk, xr)
    iota = lax.broadcasted_iota(jnp.int32, (R, N), 1)

    zr = jnp.zeros((R, _D - 72), jnp.float32)
    tab_out[0] = jnp.concatenate([fr, xr, zr], axis=1)

    _pick_and_mask(d_ref, iota, N)
    kio = lax.broadcasted_iota(jnp.int32, (R, _K), 1)

    def body(j, iv):
        _, pick = _pick_and_mask(d_ref, iota, N)
        return jnp.where(kio == j, pick + b * N, iv)

    iv = lax.fori_loop(0, _K, body, jnp.zeros((R, _K), jnp.int32))
    idx_out[0] = iv


def _call_topk(xk, f, R):
    B, N, _ = xk.shape
    body = functools.partial(_topk_body, R=R, N=N)
    return pl.pallas_call(
        body,
        grid=(B, N // R),
        in_specs=[
            pl.BlockSpec((1, N, 8), lambda b, rb: (b, 0, 0)),
            pl.BlockSpec((1, N, 64), lambda b, rb: (b, 0, 0)),
        ],
        out_specs=[
            pl.BlockSpec((1, R, _K), lambda b, rb: (b, rb, 0)),
            pl.BlockSpec((1, R, _D), lambda b, rb: (b, rb, 0)),
        ],
        out_shape=[
            jax.ShapeDtypeStruct((B, N, _K), jnp.int32),
            jax.ShapeDtypeStruct((B, N, _D), jnp.float32),
        ],
        scratch_shapes=[pltpu.VMEM((R, N), jnp.float32)],
        compiler_params=pltpu.CompilerParams(
            dimension_semantics=("parallel", "arbitrary")),
    )(xk, f)


def _gather_rows(table, idx):
    (nidx,) = idx.shape
    info = plsc.get_sparse_core_info()
    nw = info.num_cores * info.num_subcores
    ch = 128
    nch = nidx // (nw * ch)
    idx3 = idx.reshape(nw, nch, ch)
    mesh = plsc.VectorSubcoreMesh(core_axis_name="c", subcore_axis_name="s")

    @functools.partial(
        pl.kernel, mesh=mesh,
        out_type=jax.ShapeDtypeStruct((nidx, _D), jnp.float32),
        scratch_types=[
            pltpu.VMEM((nch, ch), jnp.int32),
            pltpu.VMEM((ch, _D), jnp.float32),
            pltpu.SemaphoreType.DMA,
        ],
    )
    def k(tab_hbm, idx_hbm, out_hbm, idx_v, rows_v, sem):
        wid = lax.axis_index("s") * info.num_cores + lax.axis_index("c")
        base = wid * nch * ch
        pltpu.sync_copy(idx_hbm.at[wid], idx_v)

        def body(j, _):
            pltpu.async_copy(tab_hbm.at[idx_v.at[j]], rows_v, sem).wait()
            pltpu.sync_copy(rows_v, out_hbm.at[pl.ds(base + j * ch, ch)])
            return 0

        lax.fori_loop(0, nch, body, 0)

    return k(table, idx3)


def _edge_body(xk_ref, f_ref, gath_ref, ewT_ref,
               xwT_ref, fwT_ref, misc_ref, xs_out_ref, f_out_ref, *, R):
    xr = xk_ref[0]
    fr = f_ref[0]
    eb = misc_ref[0:1, :]
    xb = misc_ref[2:3, 0:8]
    fb = misc_ref[3:4, :]

    s = jnp.zeros((R, 64), jnp.float32)
    xacc = jnp.zeros((R, 8), jnp.float32)
    zr7 = jnp.zeros((R, 7), jnp.float32)
    for k in range(_K):
        gk = gath_ref[0, :, k, :]
        fd = gk[:, 0:64] - fr
        xd = gk[:, 64:72] - xr
        xdsq = _sum_sq_seq(xd, 6)
        feat = jnp.concatenate([fd, fr, xdsq, zr7], axis=1)
        mj = _silu(jnp.dot(feat, ewT_ref[...],
                           preferred_element_type=jnp.float32) + eb)
        s = s + mj
        phi = _silu(jnp.dot(mj, xwT_ref[...],
                            preferred_element_type=jnp.float32) + xb)
        p0 = phi[:, 0:1]
        p1 = phi[:, 1:2]
        phic = jnp.concatenate([p0, p0, p0, p1, p1, p1, p0, p0], axis=1)
        xacc = xacc + xd * phic

    xs_out_ref[0] = xr + xacc * (1.0 / _K)
    f_in = jnp.concatenate([fr, s], axis=1)
    f_out_ref[0] = _silu(jnp.dot(f_in, fwT_ref[...],
                                 preferred_element_type=jnp.float32) + fb)


def _call_edge(xk, f, gath, ewT, xwT, fwT, misc, R):
    B, N, _ = xk.shape
    body = functools.partial(_edge_body, R=R)
    return pl.pallas_call(
        body,
        grid=(B, N // R),
        in_specs=[
            pl.BlockSpec((1, R, 8), lambda b, rb: (b, rb, 0)),
            pl.BlockSpec((1, R, 64), lambda b, rb: (b, rb, 0)),
            pl.BlockSpec((1, R, _K, _D), lambda b, rb: (b, rb, 0, 0)),
            pl.BlockSpec((136, 64), lambda b, rb: (0, 0)),
            pl.BlockSpec((64, 8), lambda b, rb: (0, 0)),
            pl.BlockSpec((128, 64), lambda b, rb: (0, 0)),
            pl.BlockSpec((8, 64), lambda b, rb: (0, 0)),
        ],
        out_specs=[
            pl.BlockSpec((1, R, 8), lambda b, rb: (b, rb, 0)),
            pl.BlockSpec((1, R, 64), lambda b, rb: (b, rb, 0)),
        ],
        out_shape=[
            jax.ShapeDtypeStruct((B, N, 8), jnp.float32),
            jax.ShapeDtypeStruct((B, N, 64), jnp.float32),
        ],
        compiler_params=pltpu.CompilerParams(
            dimension_semantics=("parallel", "arbitrary")),
    )(xk, f, gath, ewT, xwT, fwT, misc)


def _pad_cols(a, n):
    return jnp.pad(a, ((0, 0), (0, n - a.shape[1])))


def _misc(row0, row1, xb, fb):
    z = jnp.zeros((64,), jnp.float32)
    xbp = jnp.pad(xb, (0, 64 - xb.shape[0]))
    return jnp.stack([row0, row1, xbp, fb, z, z, z, z], axis=0)


def kernel(pts, params):
    B, _, N = pts.shape
    R = 256 if N % 256 == 0 else N // 2
    p = params

    xk = jnp.transpose(pts[:, :3, :], (0, 2, 1))
    xk = jnp.pad(xk, ((0, 0), (0, 0), (0, 5)))

    misc1 = _misc(p['e1_b'], p['e1_w'][:, 2], p['x1_b'], p['f1_b'])
    e1wT = jnp.pad(p['e1_w'].T, ((0, 5), (0, 0)))
    xwT1 = _pad_cols(p['x1_w'].T, 8)
    f1wT = jnp.pad(p['f1_w'].T, ((0, 7), (0, 0)))
    f0 = jnp.ones((B, N, 64), jnp.float32)
    idx1, tab1 = _call_topk(xk, f0, R)
    gath1 = _gather_rows(tab1.reshape(B * N, _D),
                         idx1.reshape(B * N * _K)).reshape(B, N, _K, _D)
    xs, f = _call_edge1(xk, gath1, e1wT, xwT1, f1wT, misc1, R)

    for l in (2, 3, 4):
        ew = p[f'e{l}_w']
        ewT = jnp.pad(ew.T, ((0, 7), (0, 0)))
        xwT = _pad_cols(p[f'x{l}_w'].T, 8)
        fwT = p[f'f{l}_w'].T
        misc = _misc(p[f'e{l}_b'], ew[:, 128], p[f'x{l}_b'], p[f'f{l}_b'])

        idx, tab = _call_topk(xs, f, R)
        gath = _gather_rows(tab.reshape(B * N, _D),
                            idx.reshape(B * N * _K))
        gath = gath.reshape(B, N, _K, _D)
        xs, f = _call_edge(xs, f, gath, ewT, xwT, fwT, misc, R)

    x_out = jnp.transpose(xs[:, :, 0:6], (0, 2, 1))
    f_out = jnp.transpose(f, (0, 2, 1))
    return x_out, f_out
